# baseline (speedup 1.0000x reference)
import functools

import jax
import jax.numpy as jnp
from jax import lax
from jax.experimental import pallas as pl
from jax.experimental.pallas import tpu as pltpu
from jax.experimental.pallas import tpu_sc as plsc

N_EXP = 16
BLOCK_M = 1024
LANES = 16
N_SUB = 32


def _matmul_block(x_ref, w_ref, lt_ref):
    lt_ref[...] = jax.lax.dot_general(
        w_ref[...], x_ref[...], (((1,), (1,)), ((), ())),
        preferred_element_type=jnp.float32)


def _logits_t(x_flat, W):
    n_tok, d = x_flat.shape
    grid = (n_tok // BLOCK_M,)
    return pl.pallas_call(
        _matmul_block,
        grid=grid,
        in_specs=[
            pl.BlockSpec((BLOCK_M, d), lambda i: (i, 0)),
            pl.BlockSpec((N_EXP, d), lambda i: (0, 0)),
        ],
        out_specs=pl.BlockSpec((N_EXP, BLOCK_M), lambda i: (0, i)),
        out_shape=jax.ShapeDtypeStruct((N_EXP, n_tok), jnp.float32),
    )(x_flat, W)


def _make_router(n_tok):
    chunk = n_tok // N_SUB
    n_grp = chunk // LANES
    mesh = plsc.VectorSubcoreMesh(core_axis_name="c", subcore_axis_name="s")

    @functools.partial(
        pl.kernel,
        out_type=[
            jax.ShapeDtypeStruct((2, n_tok), jnp.float32),
            jax.ShapeDtypeStruct((2, n_tok), jnp.int32),
            jax.ShapeDtypeStruct((N_SUB, 2, N_EXP, LANES), jnp.float32),
        ],
        mesh=mesh,
        scratch_types=[
            pltpu.VMEM((N_EXP, n_tok // N_SUB), jnp.float32),
            pltpu.VMEM((2, n_tok // N_SUB), jnp.float32),
            pltpu.VMEM((2, n_tok // N_SUB), jnp.int32),
            pltpu.VMEM((2, N_EXP, LANES), jnp.float32),
            pltpu.SemaphoreType.DMA,
        ],
    )
    def router(lt_hbm, w_hbm, i_hbm, fp_hbm, lt_v, w_v, i_v, fp_v, sem):
        wid = lax.axis_index("s") * 2 + lax.axis_index("c")
        base = wid * chunk
        copy = pltpu.make_async_copy(lt_hbm.at[:, pl.ds(base, chunk)], lt_v, sem)
        copy.start()

        zeros = jnp.zeros((LANES,), jnp.float32)
        izeros = jnp.zeros((LANES,), jnp.int32)
        iones = jnp.ones((LANES,), jnp.int32)

        copy.wait()

        def count_packed(accs, idx):
            fld = (idx & 3) << 3
            bit = iones << fld
            hi = idx >> 2
            return tuple(
                accs[k] + jnp.where(hi == k, bit, izeros) for k in range(4))

        def group(g, carry):
            accp, accf = carry
            sl = pl.ds(g * LANES, LANES)
            ee = [jnp.exp(lt_v[e, sl]) for e in range(N_EXP)]
            s = ee[0]
            for e in range(1, N_EXP):
                s = s + ee[e]
            inv = 1.0 / s
            max1 = ee[0]
            idx1 = izeros
            max2 = jnp.full((LANES,), -1.0, jnp.float32)
            idx2 = izeros
            for e in range(1, N_EXP):
                cur = ee[e]
                eidx = jnp.full((LANES,), e, jnp.int32)
                gt1 = cur > max1
                gt2 = cur > max2
                max2 = jnp.where(gt1, max1, jnp.where(gt2, cur, max2))
                idx2 = jnp.where(gt1, idx1, jnp.where(gt2, eidx, idx2))
                max1 = jnp.where(gt1, cur, max1)
                idx1 = jnp.where(gt1, eidx, idx1)
            tot = 1.0 / (max1 + max2)
            w_v[0, sl] = max1 * tot
            w_v[1, sl] = max2 * tot
            i_v[0, sl] = idx1
            i_v[1, sl] = idx2
            accf = count_packed(accf, idx1)
            accf = count_packed(accf, idx2)
            accp = tuple(accp[e] + ee[e] * inv for e in range(N_EXP))
            return (accp, accf)

        accp0 = tuple(zeros for _ in range(N_EXP))
        accf0 = tuple(izeros for _ in range(4))
        accp, accf = lax.fori_loop(0, n_grp, group, (accp0, accf0))

        for e in range(N_EXP):
            cnt = (accf[e >> 2] >> ((e & 3) << 3)) & 255
            fp_v[0, e, :] = cnt.astype(jnp.float32)
            fp_v[1, e, :] = accp[e]

        pltpu.sync_copy(w_v, w_hbm.at[:, pl.ds(base, chunk)])
        pltpu.sync_copy(i_v, i_hbm.at[:, pl.ds(base, chunk)])
        pltpu.sync_copy(fp_v, fp_hbm.at[wid])

    return router


N_CHUNK = 4


def kernel(x, W):
    b, t, d = x.shape
    n_tok = b * t
    x_flat = x.reshape(n_tok, d)

    tc = n_tok // N_CHUNK
    router = _make_router(tc)
    ws,is_, fps = [], [], []
    for c in range(N_CHUNK):
        ltc = _logits_t(x_flat[c * tc:(c + 1) * tc], W)
        wc, ic, fpc = router(ltc)
        ws.append(wc)
        is_.append(ic)
        fps.append(fpc)
    wout = jnp.concatenate(ws, axis=1)
    iout = jnp.concatenate(is_, axis=1)
    fp = jnp.stack(fps)

    f_i = fp[:, :, 0, :, :].sum(axis=(0, 1, 3)) / n_tok
    p_i = fp[:, :, 1, :, :].sum(axis=(0, 1, 3)) / n_tok
    aux_loss = N_EXP * jnp.sum(f_i * p_i)
    return (wout.T, iout.T, aux_loss)


# speedup vs baseline: 2.5557x; 2.5557x over previous
import jax
import jax.numpy as jnp
from jax.experimental import pallas as pl

N_EXP = 16
BLOCK_M = 1024


def _router_block(x_ref, w_ref, wout_ref, iout_ref, aux_ref):
    xb = x_ref[...]
    wt = w_ref[...]
    lt = jax.lax.dot_general(
        wt, xb, (((1,), (1,)), ((), ())),
        preferred_element_type=jnp.float32)
    e = jnp.exp(lt)
    s = jnp.sum(e, axis=0)

    iota = jax.lax.broadcasted_iota(jnp.int32, (N_EXP, BLOCK_M), 0)
    p1 = jnp.max(e, axis=0)
    i1 = jnp.min(jnp.where(e == p1[None, :], iota, N_EXP), axis=0)
    masked = jnp.where(iota == i1[None, :], -1.0, e)
    p2 = jnp.max(masked, axis=0)
    i2 = jnp.min(jnp.where(masked == p2[None, :], iota, N_EXP), axis=0)

    tot = p1 + p2
    wout_ref[...] = jnp.stack([p1 / tot, p2 / tot], axis=-1)
    iout_ref[...] = jnp.stack([i1, i2], axis=-1)

    sel = (jnp.where(iota == i1[None, :], 1.0, 0.0)
           + jnp.where(iota == i2[None, :], 1.0, 0.0))
    aux_ref[0, 0, :] = jnp.sum(sel, axis=1)
    aux_ref[0, 1, :] = jnp.sum(e / s[None, :], axis=1)


def kernel(x, W):
    b, t, d = x.shape
    n_tok = b * t
    x_flat = x.reshape(n_tok, d)
    grid = (n_tok // BLOCK_M,)

    wout, iout, aux = pl.pallas_call(
        _router_block,
        grid=grid,
        in_specs=[
            pl.BlockSpec((BLOCK_M, d), lambda i: (i, 0)),
            pl.BlockSpec((N_EXP, d), lambda i: (0, 0)),
        ],
        out_specs=[
            pl.BlockSpec((BLOCK_M, 2), lambda i: (i, 0)),
            pl.BlockSpec((BLOCK_M, 2), lambda i: (i, 0)),
            pl.BlockSpec((1, 2, N_EXP), lambda i: (i, 0, 0)),
        ],
        out_shape=[
            jax.ShapeDtypeStruct((n_tok, 2), jnp.float32),
            jax.ShapeDtypeStruct((n_tok, 2), jnp.int32),
            jax.ShapeDtypeStruct((grid[0], 2, N_EXP), jnp.float32),
        ],
    )(x_flat, W)

    f_i = aux[:, 0, :].sum(axis=0) / n_tok
    p_i = aux[:, 1, :].sum(axis=0) / n_tok
    aux_loss = N_EXP * jnp.sum(f_i * p_i)
    return (wout, iout, aux_loss)


# speedup vs baseline: 3.5079x; 1.3726x over previous
import jax
import jax.numpy as jnp
from jax.experimental import pallas as pl

N_EXP = 16
BLOCK_M = 1024


def _router_block(x1_ref, x2_ref, w_ref, wout_ref, iout_ref, aux_ref):
    wt = w_ref[...]
    hd = x1_ref.shape[1]
    lt = jax.lax.dot_general(
        wt[:, :hd], x1_ref[...], (((1,), (1,)), ((), ())),
        preferred_element_type=jnp.float32)
    lt = lt + jax.lax.dot_general(
        wt[:, hd:], x2_ref[...], (((1,), (1,)), ((), ())),
        preferred_element_type=jnp.float32)
    e = jnp.exp(lt)
    s = jnp.sum(e, axis=0)

    iota = jax.lax.broadcasted_iota(jnp.int32, (N_EXP, BLOCK_M), 0)
    p1 = jnp.max(e, axis=0)
    i1 = jnp.min(jnp.where(e == p1[None, :], iota, N_EXP), axis=0)
    masked = jnp.where(iota == i1[None, :], -1.0, e)
    p2 = jnp.max(masked, axis=0)
    i2 = jnp.min(jnp.where(masked == p2[None, :], iota, N_EXP), axis=0)

    tot = p1 + p2
    wout_ref[0, :] = p1 / tot
    wout_ref[1, :] = p2 / tot
    iout_ref[0, :] = i1
    iout_ref[1, :] = i2

    sel = (jnp.where(iota == i1[None, :], 1.0, 0.0)
           + jnp.where(iota == i2[None, :], 1.0, 0.0))
    aux_ref[0, 0, :] = jnp.sum(sel, axis=1)
    aux_ref[0, 1, :] = jnp.sum(e / s[None, :], axis=1)


def kernel(x, W):
    b, t, d = x.shape
    n_tok = b * t
    x_flat = x.reshape(n_tok, d)
    grid = (n_tok // BLOCK_M,)

    wout, iout, aux = pl.pallas_call(
        _router_block,
        grid=grid,
        in_specs=[
            pl.BlockSpec((BLOCK_M, d // 2), lambda i: (i, 0)),
            pl.BlockSpec((BLOCK_M, d // 2), lambda i: (i, 1)),
            pl.BlockSpec((N_EXP, d), lambda i: (0, 0)),
        ],
        out_specs=[
            pl.BlockSpec((2, BLOCK_M), lambda i: (0, i)),
            pl.BlockSpec((2, BLOCK_M), lambda i: (0, i)),
            pl.BlockSpec((1, 2, N_EXP), lambda i: (i, 0, 0)),
        ],
        out_shape=[
            jax.ShapeDtypeStruct((2, n_tok), jnp.float32),
            jax.ShapeDtypeStruct((2, n_tok), jnp.int32),
            jax.ShapeDtypeStruct((grid[0], 2, N_EXP), jnp.float32),
        ],
    )(x_flat, x_flat, W)

    f_i = aux[:, 0, :].sum(axis=0) / n_tok
    p_i = aux[:, 1, :].sum(axis=0) / n_tok
    aux_loss = N_EXP * jnp.sum(f_i * p_i)
    return (wout.T, iout.T, aux_loss)


# speedup vs baseline: 3.5488x; 1.0116x over previous
import jax
import jax.numpy as jnp
from jax.experimental import pallas as pl

N_EXP = 16
BLOCK_M = 1024


def _router_block(x_ref, w_ref, wout_ref, iout_ref, aux_ref):
    xb = x_ref[...]
    wt = w_ref[...]
    lt = jax.lax.dot_general(
        wt, xb, (((1,), (1,)), ((), ())),
        preferred_element_type=jnp.float32)
    e = jnp.exp(lt)
    s = jnp.sum(e, axis=0)

    iota = jax.lax.broadcasted_iota(jnp.int32, (N_EXP, BLOCK_M), 0)
    p1 = jnp.max(e, axis=0)
    i1 = jnp.min(jnp.where(e == p1[None, :], iota, N_EXP), axis=0)
    masked = jnp.where(iota == i1[None, :], -1.0, e)
    p2 = jnp.max(masked, axis=0)
    i2 = jnp.min(jnp.where(masked == p2[None, :], iota, N_EXP), axis=0)

    tot = p1 + p2
    wout_ref[0, :] = p1 / tot
    wout_ref[1, :] = p2 / tot
    iout_ref[0, :] = i1
    iout_ref[1, :] = i2

    sel = (jnp.where(iota == i1[None, :], 1.0, 0.0)
           + jnp.where(iota == i2[None, :], 1.0, 0.0))
    aux_ref[0, 0, :] = jnp.sum(sel, axis=1)
    aux_ref[0, 1, :] = jnp.sum(e / s[None, :], axis=1)


def kernel(x, W):
    b, t, d = x.shape
    n_tok = b * t
    x_flat = x.reshape(n_tok, d)
    grid = (n_tok // BLOCK_M,)

    wout, iout, aux = pl.pallas_call(
        _router_block,
        grid=grid,
        in_specs=[
            pl.BlockSpec((BLOCK_M, d), lambda i: (i, 0)),
            pl.BlockSpec((N_EXP, d), lambda i: (0, 0)),
        ],
        out_specs=[
            pl.BlockSpec((2, BLOCK_M), lambda i: (0, i)),
            pl.BlockSpec((2, BLOCK_M), lambda i: (0, i)),
            pl.BlockSpec((1, 2, N_EXP), lambda i: (i, 0, 0)),
        ],
        out_shape=[
            jax.ShapeDtypeStruct((2, n_tok), jnp.float32),
            jax.ShapeDtypeStruct((2, n_tok), jnp.int32),
            jax.ShapeDtypeStruct((grid[0], 2, N_EXP), jnp.float32),
        ],
    )(x_flat, W)

    f_i = aux[:, 0, :].sum(axis=0) / n_tok
    p_i = aux[:, 1, :].sum(axis=0) / n_tok
    aux_loss = N_EXP * jnp.sum(f_i * p_i)
    return (wout.T, iout.T, aux_loss)
